# baseline (speedup 1.0000x reference)
import jax
import jax.numpy as jnp
from jax.experimental import pallas as pl
from jax.experimental.pallas import tpu as pltpu

_B, _S, _P, _N = 32, 2048, 256, 256
_K, _L = 8, 4
_NB = 4
_NSTEPS = _B // _NB
_EPS = 1e-5


def _main_kernel(benc_ref, wb_ref, wenc_ref, x_hbm, o_hbm,
                 g_ref, rs_ref, xbuf, obuf, xsem, osem):
    pltpu.make_async_copy(
        x_hbm.at[0:_NB], xbuf.at[0], xsem.at[0]).start()
    pltpu.make_async_copy(
        x_hbm.at[_NB:2 * _NB], xbuf.at[1], xsem.at[1]).start()

    def mm(a, b):
        return jnp.dot(a, b, precision=jax.lax.Precision.HIGHEST,
                       preferred_element_type=jnp.float32)

    w1 = wb_ref[...]
    w2 = mm(w1, w1)
    w3 = mm(w2, w1)
    w4 = mm(w2, w2)
    w5 = mm(w4, w1)
    w6 = mm(w4, w2)
    w7 = mm(w4, w3)
    w8 = mm(w4, w4)
    w9 = mm(w8, w1)
    w10 = mm(w8, w2)
    w11 = mm(w8, w3)
    pows = [w1, w2, w3, w4, w5, w6, w7, w8, w9, w10, w11]
    for l in range(_L):
        for k in range(_K):
            m = _K - k + l
            g_ref[l * _P:(l + 1) * _P, k * _P:(k + 1) * _P] = (
                wenc_ref[l, k] * pows[m - 1])
    rs = jnp.sum(g_ref[...], axis=1, keepdims=True)
    rs_ref[...] = (1.0 - rs) * jnp.ones((1, _N), jnp.float32)

    for i in range(_NSTEPS):
        slot = i % 2
        pltpu.make_async_copy(
            x_hbm.at[i * _NB:(i + 1) * _NB], xbuf.at[slot],
            xsem.at[slot]).wait()
        if i >= 2:
            pltpu.make_async_copy(
                obuf.at[slot], o_hbm.at[(i - 2) * _NB:(i - 1) * _NB],
                osem.at[slot]).wait()
        for b in range(_NB):
            x = xbuf[slot, b]
            s1 = jnp.sum(x, axis=0, keepdims=True)
            s2 = jnp.sum(x * x, axis=0, keepdims=True)
            mean = s1 * (1.0 / _S)
            var = s2 * (1.0 / _S) - mean * mean
            stdev = jnp.sqrt(var + _EPS)
            d = jnp.dot(g_ref[...], x, preferred_element_type=jnp.float32)
            for l in range(_L):
                obuf[slot, b, l * _P:(l + 1) * _P, :] = (
                    d[l * _P:(l + 1) * _P, :]
                    + (rs_ref[l * _P:(l + 1) * _P, :] * mean
                       + benc_ref[l] * stdev))
        pltpu.make_async_copy(
            obuf.at[slot], o_hbm.at[i * _NB:(i + 1) * _NB],
            osem.at[slot]).start()
        if i + 2 < _NSTEPS:
            pltpu.make_async_copy(
                x_hbm.at[(i + 2) * _NB:(i + 3) * _NB], xbuf.at[slot],
                xsem.at[slot]).start()

    pltpu.make_async_copy(
        obuf.at[0], o_hbm.at[(_NSTEPS - 2) * _NB:(_NSTEPS - 1) * _NB],
        osem.at[0]).wait()
    pltpu.make_async_copy(
        obuf.at[1], o_hbm.at[(_NSTEPS - 1) * _NB:_NSTEPS * _NB],
        osem.at[1]).wait()


def kernel(x_enc, x_mark_enc, x_dec, x_mark_dec, W_base, W_enc, b_enc):
    out = pl.pallas_call(
        _main_kernel,
        grid=(),
        out_shape=jax.ShapeDtypeStruct((_B, _L * _P, _N), jnp.float32),
        in_specs=[
            pl.BlockSpec(memory_space=pltpu.SMEM),
            pl.BlockSpec(memory_space=pltpu.VMEM),
            pl.BlockSpec(memory_space=pltpu.SMEM),
            pl.BlockSpec(memory_space=pl.ANY),
        ],
        out_specs=pl.BlockSpec(memory_space=pl.ANY),
        scratch_shapes=[
            pltpu.VMEM((_L * _P, _K * _P), jnp.float32),
            pltpu.VMEM((_L * _P, _N), jnp.float32),
            pltpu.VMEM((2, _NB, _S, _N), jnp.float32),
            pltpu.VMEM((2, _NB, _L * _P, _N), jnp.float32),
            pltpu.SemaphoreType.DMA((2,)),
            pltpu.SemaphoreType.DMA((2,)),
        ],
        name="seg_linear_manual",
    )(b_enc, W_base, W_enc, x_enc)
    return out
